# baseline (speedup 1.0000x reference)
import dataclasses
import functools

import jax
import jax.numpy as jnp
from jax import lax
from jax.experimental import pallas as pl
from jax.experimental.pallas import tpu as pltpu
from jax.experimental.pallas import tpu_sc as plsc


def _tc_kernel(st_ref, a_ref, rmf_ref, rows_ref, hv_ref):
    p = st_ref[0]
    px = p[:, 0:1]
    py = p[:, 1:2]
    pz = p[:, 2:3]
    ax = a_ref[0, 0:1, :]
    ay = a_ref[0, 1:2, :]
    az = a_ref[0, 2:3, :]
    rmf = rmf_ref[0]

    NSP = ax.shape[1]
    T = p.shape[0]

    z1 = jnp.zeros((1, 1), jnp.float32)
    bx = jnp.concatenate([ax[:, 1:], z1], axis=1)
    by = jnp.concatenate([ay[:, 1:], z1], axis=1)
    bz = jnp.concatenate([az[:, 1:], z1], axis=1)

    rmf_next = jnp.concatenate([rmf[:, 1:], jnp.zeros((1, 1), jnp.float32)],
                               axis=1)
    lane = jax.lax.broadcasted_iota(jnp.int32, (1, NSP), 1)
    lane_ok = jnp.where((lane & 127) != 127, 1.0, 0.0)
    m = rmf * rmf_next * lane_ok
    mbig = (1.0 - m) * 1e30

    dx = bx - ax
    dy = by - ay
    dz = bz - az
    dd = jnp.maximum(dx * dx + dy * dy + dz * dz, 1e-12)
    rdd = 1.0 / dd

    ex = px - ax
    ey = py - ay
    ez = pz - az
    tn = ex * dx + ey * dy + ez * dz
    pa2 = ex * ex + ey * ey + ez * ez
    t = jnp.clip(tn * rdd, 0.0, 1.0)
    dist2 = pa2 + t * (t * dd - tn - tn) + mbig

    best = jnp.argmin(dist2, axis=1).astype(jnp.int32)
    rows_ref[0] = (best * 3)[None, :]
    hv = jnp.any(m > 0.0).astype(jnp.float32)
    hv_ref[0] = jnp.broadcast_to(hv, (1, T))


def _tc_run(st, rpT, rmf):
    N, T, F = st.shape
    NSP = rpT.shape[2]
    return pl.pallas_call(
        _tc_kernel,
        grid=(N,),
        in_specs=[
            pl.BlockSpec((1, T, F), lambda n: (n, 0, 0)),
            pl.BlockSpec((1, 3, NSP), lambda n: (n, 0, 0)),
            pl.BlockSpec((1, 1, NSP), lambda n: (n, 0, 0)),
        ],
        out_specs=[
            pl.BlockSpec((1, 1, T), lambda n: (n, 0, 0)),
            pl.BlockSpec((1, 1, T), lambda n: (n, 0, 0)),
        ],
        out_shape=[
            jax.ShapeDtypeStruct((N, 1, T), jnp.int32),
            jax.ShapeDtypeStruct((N, 1, T), jnp.float32),
        ],
    )(st, rpT, rmf)


def _make_sc_kernel(T, F):
    L = 16

    def _sc_kernel(rp_ref, rows_ref, st_ref, hv_ref, out_ref,
                   table_v, idx_v, st_v, hvrow_v):
        wid = lax.axis_index("s") * 2 + lax.axis_index("c")
        for k in range(2):
            n = wid * 2 + k
            pltpu.sync_copy(rp_ref.at[n], table_v)
            pltpu.sync_copy(rows_ref.at[n], idx_v)
            pltpu.sync_copy(st_ref.at[n], st_v)
            pltpu.sync_copy(hv_ref.at[n], hvrow_v)
            for c in range(T // L):
                sl = pl.ds(c * L, L)
                r3 = idx_v[0, sl]
                ax = plsc.load_gather(table_v, [r3])
                ay = plsc.load_gather(table_v, [r3 + 1])
                az = plsc.load_gather(table_v, [r3 + 2])
                bx = plsc.load_gather(table_v, [r3 + 3])
                by = plsc.load_gather(table_v, [r3 + 4])
                bz = plsc.load_gather(table_v, [r3 + 5])
                t6 = (lax.broadcasted_iota(jnp.int32, (L,), 0) + c * L) * F
                px = plsc.load_gather(st_v, [t6])
                py = plsc.load_gather(st_v, [t6 + 1])
                pz = plsc.load_gather(st_v, [t6 + 2])
                dx = bx - ax
                dy = by - ay
                dz = bz - az
                dd = jnp.maximum(dx * dx + dy * dy + dz * dz, 1e-12)
                tn = (px - ax) * dx + (py - ay) * dy + (pz - az) * dz
                t = jnp.clip(tn / dd, 0.0, 1.0)
                qx = ax + t * dx
                qy = ay + t * dy
                qz = az + t * dz
                hv = hvrow_v[0, sl] > 0.0
                plsc.store_scatter(st_v, [t6], jnp.where(hv, qx, px))
                plsc.store_scatter(st_v, [t6 + 1], jnp.where(hv, qy, py))
                plsc.store_scatter(st_v, [t6 + 2], jnp.where(hv, qz, pz))
            pltpu.sync_copy(st_v, out_ref.at[n])

    return _sc_kernel


def _sc_run(rp_flat, rows, st_flat, hvb):
    N = rows.shape[0]
    T = rows.shape[2]
    F = st_flat.shape[1] // T
    mesh = plsc.VectorSubcoreMesh(core_axis_name="c", subcore_axis_name="s",
                                  num_cores=2, num_subcores=16)
    cp = pltpu.CompilerParams()
    if "needs_layout_passes" in pltpu.CompilerParams.__dataclass_fields__:
        cp = dataclasses.replace(cp, needs_layout_passes=False)
    k = pl.kernel(
        _make_sc_kernel(T, F),
        out_type=jax.ShapeDtypeStruct((N, T * F), jnp.float32),
        mesh=mesh,
        scratch_types=[
            pltpu.VMEM((rp_flat.shape[1],), jnp.float32),
            pltpu.VMEM((1, T), jnp.int32),
            pltpu.VMEM((T * F,), jnp.float32),
            pltpu.VMEM((1, T), jnp.float32),
        ],
        compiler_params=cp,
    )
    return k(rp_flat, rows, st_flat, hvb)


@jax.jit
def _run(selected_traj, road_points, road_mask):
    N, T, F = selected_traj.shape
    _, NB, NP, D = road_points.shape
    NSP = NB * NP

    st = selected_traj.astype(jnp.float32)
    rpT = road_points.transpose(0, 3, 1, 2).reshape(N, D, NSP)
    rp_flat = road_points.reshape(N, NSP * D)
    rmf = road_mask.reshape(N, 1, NSP).astype(jnp.float32)

    rows, hvb = _tc_run(st, rpT, rmf)
    out = _sc_run(rp_flat, rows, st.reshape(N, T * F), hvb)
    return out.reshape(N, T, F)


def kernel(selected_traj, road_points, road_mask):
    return _run(selected_traj, road_points, road_mask)


# speedup vs baseline: 1.0329x; 1.0329x over previous
import dataclasses
import functools

import jax
import jax.numpy as jnp
from jax import lax
from jax.experimental import pallas as pl
from jax.experimental.pallas import tpu as pltpu
from jax.experimental.pallas import tpu_sc as plsc


_ROWS_PER_STEP = 2


def _tc_kernel(st_ref, a_ref, rmf_ref, rows_ref):
    NSP = a_ref.shape[2]
    T = st_ref.shape[1]
    lane = jax.lax.broadcasted_iota(jnp.int32, (1, NSP), 1)
    lane_ok = jnp.where((lane & 127) != 127, 1.0, 0.0)
    z1 = jnp.zeros((1, 1), jnp.float32)

    for i in range(_ROWS_PER_STEP):
        p = st_ref[i]
        px = p[:, 0:1]
        py = p[:, 1:2]
        pz = p[:, 2:3]
        ax = a_ref[i, 0:1, :]
        ay = a_ref[i, 1:2, :]
        az = a_ref[i, 2:3, :]
        rmf = rmf_ref[i]

        bx = jnp.concatenate([ax[:, 1:], z1], axis=1)
        by = jnp.concatenate([ay[:, 1:], z1], axis=1)
        bz = jnp.concatenate([az[:, 1:], z1], axis=1)

        rmf_next = jnp.concatenate([rmf[:, 1:], z1], axis=1)
        m = rmf * rmf_next * lane_ok
        mbig = (1.0 - m) * 1e30

        dx = bx - ax
        dy = by - ay
        dz = bz - az
        dd = jnp.maximum(dx * dx + dy * dy + dz * dz, 1e-12)
        rdd = 1.0 / dd

        ex = px - ax
        ey = py - ay
        ez = pz - az
        tn = ex * dx + ey * dy + ez * dz
        pa2 = ex * ex + ey * ey + ez * ez
        t = jnp.clip(tn * rdd, 0.0, 1.0)
        dist2 = pa2 + t * (t * dd - tn - tn) + mbig

        best = jnp.argmin(dist2, axis=1).astype(jnp.int32)
        hv = jnp.any(m > 0.0)
        enc = jnp.where(hv, best * 3, -1)
        rows_ref[i] = enc[None, :]


def _tc_run(st, rpT, rmf):
    N, T, F = st.shape
    NSP = rpT.shape[2]
    R = _ROWS_PER_STEP
    return pl.pallas_call(
        _tc_kernel,
        grid=(N // R,),
        in_specs=[
            pl.BlockSpec((R, T, F), lambda n: (n, 0, 0)),
            pl.BlockSpec((R, 3, NSP), lambda n: (n, 0, 0)),
            pl.BlockSpec((R, 1, NSP), lambda n: (n, 0, 0)),
        ],
        out_specs=pl.BlockSpec((R, 1, T), lambda n: (n, 0, 0)),
        out_shape=jax.ShapeDtypeStruct((N, 1, T), jnp.int32),
    )(st, rpT, rmf)


def _make_sc_kernel(T, F):
    L = 16

    def _sc_kernel(rp_ref, rows_ref, st_ref, out_ref,
                   table_v, idx_v, st_v):
        wid = lax.axis_index("s") * 2 + lax.axis_index("c")
        for k in range(2):
            n = wid * 2 + k
            pltpu.sync_copy(rp_ref.at[n], table_v)
            pltpu.sync_copy(rows_ref.at[n], idx_v)
            pltpu.sync_copy(st_ref.at[n], st_v)
            for c in range(T // L):
                sl = pl.ds(c * L, L)
                enc = idx_v[0, sl]
                hv = enc >= 0
                r3 = jnp.maximum(enc, 0)
                ax = plsc.load_gather(table_v, [r3])
                ay = plsc.load_gather(table_v, [r3 + 1])
                az = plsc.load_gather(table_v, [r3 + 2])
                bx = plsc.load_gather(table_v, [r3 + 3])
                by = plsc.load_gather(table_v, [r3 + 4])
                bz = plsc.load_gather(table_v, [r3 + 5])
                t6 = (lax.broadcasted_iota(jnp.int32, (L,), 0) + c * L) * F
                px = plsc.load_gather(st_v, [t6])
                py = plsc.load_gather(st_v, [t6 + 1])
                pz = plsc.load_gather(st_v, [t6 + 2])
                dx = bx - ax
                dy = by - ay
                dz = bz - az
                dd = jnp.maximum(dx * dx + dy * dy + dz * dz, 1e-12)
                tn = (px - ax) * dx + (py - ay) * dy + (pz - az) * dz
                t = jnp.clip(tn / dd, 0.0, 1.0)
                qx = ax + t * dx
                qy = ay + t * dy
                qz = az + t * dz
                plsc.store_scatter(st_v, [t6], jnp.where(hv, qx, px))
                plsc.store_scatter(st_v, [t6 + 1], jnp.where(hv, qy, py))
                plsc.store_scatter(st_v, [t6 + 2], jnp.where(hv, qz, pz))
            pltpu.sync_copy(st_v, out_ref.at[n])

    return _sc_kernel


def _sc_run(rp_flat, rows, st_flat):
    N = rows.shape[0]
    T = rows.shape[2]
    F = st_flat.shape[1] // T
    mesh = plsc.VectorSubcoreMesh(core_axis_name="c", subcore_axis_name="s",
                                  num_cores=2, num_subcores=16)
    cp = pltpu.CompilerParams()
    if "needs_layout_passes" in pltpu.CompilerParams.__dataclass_fields__:
        cp = dataclasses.replace(cp, needs_layout_passes=False)
    k = pl.kernel(
        _make_sc_kernel(T, F),
        out_type=jax.ShapeDtypeStruct((N, T * F), jnp.float32),
        mesh=mesh,
        scratch_types=[
            pltpu.VMEM((rp_flat.shape[1],), jnp.float32),
            pltpu.VMEM((1, T), jnp.int32),
            pltpu.VMEM((T * F,), jnp.float32),
        ],
        compiler_params=cp,
    )
    return k(rp_flat, rows, st_flat)


@jax.jit
def _run(selected_traj, road_points, road_mask):
    N, T, F = selected_traj.shape
    _, NB, NP, D = road_points.shape
    NSP = NB * NP

    st = selected_traj.astype(jnp.float32)
    rpT = road_points.transpose(0, 3, 1, 2).reshape(N, D, NSP)
    rp_flat = road_points.reshape(N, NSP * D)
    rmf = road_mask.reshape(N, 1, NSP).astype(jnp.float32)

    rows = _tc_run(st, rpT, rmf)
    out = _sc_run(rp_flat, rows, st.reshape(N, T * F))
    return out.reshape(N, T, F)


def kernel(selected_traj, road_points, road_mask):
    return _run(selected_traj, road_points, road_mask)


# speedup vs baseline: 1.1658x; 1.1287x over previous
import dataclasses
import functools

import jax
import jax.numpy as jnp
from jax import lax
from jax.experimental import pallas as pl
from jax.experimental.pallas import tpu as pltpu
from jax.experimental.pallas import tpu_sc as plsc


_ROWS_PER_STEP = 2


def _tc_kernel(st_ref, a_ref, rmf_ref, rows_ref):
    NSP = a_ref.shape[2]
    T = st_ref.shape[1]
    lane = jax.lax.broadcasted_iota(jnp.int32, (1, NSP), 1)
    lane_ok = jnp.where((lane & 127) != 127, 1.0, 0.0)
    z1 = jnp.zeros((1, 1), jnp.float32)

    for i in range(_ROWS_PER_STEP):
        p = st_ref[i]
        P3 = p[:, 0:3]
        ax = a_ref[i, 0:1, :]
        ay = a_ref[i, 1:2, :]
        az = a_ref[i, 2:3, :]
        rmf = rmf_ref[i]

        bx = jnp.concatenate([ax[:, 1:], z1], axis=1)
        by = jnp.concatenate([ay[:, 1:], z1], axis=1)
        bz = jnp.concatenate([az[:, 1:], z1], axis=1)
        dx = bx - ax
        dy = by - ay
        dz = bz - az
        dd = jnp.maximum(dx * dx + dy * dy + dz * dz, 1e-12)
        hrdd = 0.5 / dd
        rr = ax * ax + ay * ay + az * az
        rmf_next = jnp.concatenate([rmf[:, 1:], z1], axis=1)
        m = rmf * rmf_next * lane_ok
        mbig = (1.0 - m) * 1e30

        A3 = jnp.concatenate([ax, ay, az], axis=0)
        ph = P3.astype(jnp.bfloat16)
        plo = (P3 - ph.astype(jnp.float32)).astype(jnp.bfloat16)
        ah = A3.astype(jnp.bfloat16)
        alo = (A3 - ah.astype(jnp.float32)).astype(jnp.bfloat16)
        dn = (((1,), (0,)), ((), ()))
        mm = lambda x, y: jax.lax.dot_general(
            x, y, dimension_numbers=dn,
            preferred_element_type=jnp.float32)
        pr = ((mm(ph, ah) + mm(plo, alo))
              + (mm(ph, alo) + mm(plo, ah)))
        G = rr - (pr + pr)
        Gnext = jnp.concatenate([G[:, 1:], jnp.zeros((T, 1), jnp.float32)],
                                axis=1)
        u = (G - Gnext) * hrdd + 0.5
        t = jnp.clip(u, 0.0, 1.0)
        s = t - u - u
        dist2 = G + dd * (t * s) + mbig

        best = jnp.argmin(dist2, axis=1).astype(jnp.int32)
        hv = jnp.any(m > 0.0)
        enc = jnp.where(hv, best * 3, -1)
        rows_ref[i] = enc[:, None]


def _tc_run(st, rpT, rmf):
    N, T, F = st.shape
    NSP = rpT.shape[2]
    R = _ROWS_PER_STEP
    return pl.pallas_call(
        _tc_kernel,
        grid=(N // R,),
        in_specs=[
            pl.BlockSpec((R, T, F), lambda n: (n, 0, 0)),
            pl.BlockSpec((R, 3, NSP), lambda n: (n, 0, 0)),
            pl.BlockSpec((R, 1, NSP), lambda n: (n, 0, 0)),
        ],
        out_specs=pl.BlockSpec((R, T, 1), lambda n: (n, 0, 0)),
        out_shape=jax.ShapeDtypeStruct((N, T, 1), jnp.int32),
    )(st, rpT, rmf)


def _make_sc_kernel(T, F):
    L = 16

    def _sc_kernel(rp_ref, rows_ref, st_ref, out_ref,
                   table_v, idx_v, st_v):
        wid = lax.axis_index("s") * 2 + lax.axis_index("c")
        for k in range(2):
            n = wid * 2 + k
            pltpu.sync_copy(rp_ref.at[n], table_v)
            pltpu.sync_copy(rows_ref.at[n], idx_v)
            pltpu.sync_copy(st_ref.at[n], st_v)
            for c in range(T // L):
                sl = pl.ds(c * L, L)
                tv = lax.broadcasted_iota(jnp.int32, (L,), 0) + c * L
                enc = plsc.load_gather(idx_v, [tv, tv * 0])
                hv = enc >= 0
                r3 = jnp.maximum(enc, 0)
                ax = plsc.load_gather(table_v, [r3])
                ay = plsc.load_gather(table_v, [r3 + 1])
                az = plsc.load_gather(table_v, [r3 + 2])
                bx = plsc.load_gather(table_v, [r3 + 3])
                by = plsc.load_gather(table_v, [r3 + 4])
                bz = plsc.load_gather(table_v, [r3 + 5])
                t6 = tv * F
                px = plsc.load_gather(st_v, [t6])
                py = plsc.load_gather(st_v, [t6 + 1])
                pz = plsc.load_gather(st_v, [t6 + 2])
                dx = bx - ax
                dy = by - ay
                dz = bz - az
                dd = jnp.maximum(dx * dx + dy * dy + dz * dz, 1e-12)
                tn = (px - ax) * dx + (py - ay) * dy + (pz - az) * dz
                t = jnp.clip(tn / dd, 0.0, 1.0)
                qx = ax + t * dx
                qy = ay + t * dy
                qz = az + t * dz
                plsc.store_scatter(st_v, [t6], jnp.where(hv, qx, px))
                plsc.store_scatter(st_v, [t6 + 1], jnp.where(hv, qy, py))
                plsc.store_scatter(st_v, [t6 + 2], jnp.where(hv, qz, pz))
            pltpu.sync_copy(st_v, out_ref.at[n])

    return _sc_kernel


def _sc_run(rp_flat, rows, st_flat):
    N = rows.shape[0]
    T = rows.shape[1]
    F = st_flat.shape[1] // T
    mesh = plsc.VectorSubcoreMesh(core_axis_name="c", subcore_axis_name="s",
                                  num_cores=2, num_subcores=16)
    cp = pltpu.CompilerParams()
    if "needs_layout_passes" in pltpu.CompilerParams.__dataclass_fields__:
        cp = dataclasses.replace(cp, needs_layout_passes=False)
    k = pl.kernel(
        _make_sc_kernel(T, F),
        out_type=jax.ShapeDtypeStruct((N, T * F), jnp.float32),
        mesh=mesh,
        scratch_types=[
            pltpu.VMEM((rp_flat.shape[1],), jnp.float32),
            pltpu.VMEM((T, 1), jnp.int32),
            pltpu.VMEM((T * F,), jnp.float32),
        ],
        compiler_params=cp,
    )
    return k(rp_flat, rows, st_flat)


@jax.jit
def _run(selected_traj, road_points, road_mask):
    N, T, F = selected_traj.shape
    _, NB, NP, D = road_points.shape
    NSP = NB * NP

    st = selected_traj.astype(jnp.float32)
    rpT = road_points.transpose(0, 3, 1, 2).reshape(N, D, NSP)
    rp_flat = road_points.reshape(N, NSP * D)
    rmf = road_mask.reshape(N, 1, NSP).astype(jnp.float32)

    rows = _tc_run(st, rpT, rmf)
    out = _sc_run(rp_flat, rows, st.reshape(N, T * F))
    return out.reshape(N, T, F)


def kernel(selected_traj, road_points, road_mask):
    return _run(selected_traj, road_points, road_mask)


# speedup vs baseline: 1.1969x; 1.0266x over previous
import dataclasses
import functools

import jax
import jax.numpy as jnp
from jax import lax
from jax.experimental import pallas as pl
from jax.experimental.pallas import tpu as pltpu
from jax.experimental.pallas import tpu_sc as plsc


_ROWS_PER_STEP = 2


def _tc_kernel(st_ref, a_ref, rmf_ref, rows_ref):
    NSP = a_ref.shape[2]
    T = st_ref.shape[1]
    lane = jax.lax.broadcasted_iota(jnp.int32, (1, NSP), 1)
    lane_ok = jnp.where((lane & 127) != 127, 1.0, 0.0)
    z1 = jnp.zeros((1, 1), jnp.float32)

    for i in range(_ROWS_PER_STEP):
        p = st_ref[i]
        P3 = p[:, 0:3]
        ax = a_ref[i, 0:1, :]
        ay = a_ref[i, 1:2, :]
        az = a_ref[i, 2:3, :]
        rmf = rmf_ref[i]

        bx = jnp.concatenate([ax[:, 1:], z1], axis=1)
        by = jnp.concatenate([ay[:, 1:], z1], axis=1)
        bz = jnp.concatenate([az[:, 1:], z1], axis=1)
        dx = bx - ax
        dy = by - ay
        dz = bz - az
        dd = jnp.maximum(dx * dx + dy * dy + dz * dz, 1e-12)
        hrdd = 0.5 / dd
        rr = ax * ax + ay * ay + az * az
        rmf_next = jnp.concatenate([rmf[:, 1:], z1], axis=1)
        m = rmf * rmf_next * lane_ok
        mbig = (1.0 - m) * 1e30

        A3 = jnp.concatenate([ax, ay, az], axis=0)
        ph = P3.astype(jnp.bfloat16)
        plo = (P3 - ph.astype(jnp.float32)).astype(jnp.bfloat16)
        ah = A3.astype(jnp.bfloat16)
        alo = (A3 - ah.astype(jnp.float32)).astype(jnp.bfloat16)
        dn = (((1,), (0,)), ((), ()))
        mm = lambda x, y: jax.lax.dot_general(
            x, y, dimension_numbers=dn,
            preferred_element_type=jnp.float32)
        pr = ((mm(ph, ah) + mm(plo, alo))
              + (mm(ph, alo) + mm(plo, ah)))
        G = rr - (pr + pr)
        Gnext = jnp.concatenate([G[:, 1:], jnp.zeros((T, 1), jnp.float32)],
                                axis=1)
        u = (G - Gnext) * hrdd + 0.5
        t = jnp.clip(u, 0.0, 1.0)
        s = t - u - u
        dist2 = G + dd * (t * s) + mbig

        best = jnp.argmin(dist2, axis=1).astype(jnp.int32)
        hv = jnp.any(m > 0.0)
        enc = jnp.where(hv, best * 3, -1)
        rows_ref[i] = enc[:, None]


def _tc_run(st, rpT, rmf, off, rows):
    N, T, F = st.shape
    NSP = rpT.shape[2]
    R = _ROWS_PER_STEP
    base = off // R
    return pl.pallas_call(
        _tc_kernel,
        grid=(rows // R,),
        in_specs=[
            pl.BlockSpec((R, T, F), lambda n: (n + base, 0, 0)),
            pl.BlockSpec((R, 3, NSP), lambda n: (n + base, 0, 0)),
            pl.BlockSpec((R, 1, NSP), lambda n: (n + base, 0, 0)),
        ],
        out_specs=pl.BlockSpec((R, T, 1), lambda n: (n, 0, 0)),
        out_shape=jax.ShapeDtypeStruct((rows, T, 1), jnp.int32),
    )(st, rpT, rmf)


def _make_sc_kernel(T, F, off):
    L = 16

    def _sc_kernel(rp_ref, rows_ref, st_ref, out_ref,
                   table_v, idx_v, st_v, sem1, sem2, sem3):
        wid = lax.axis_index("s") * 2 + lax.axis_index("c")
        if True:
            n = wid + off
            c1 = pltpu.async_copy(rp_ref.at[n], table_v, sem1)
            c2 = pltpu.async_copy(rows_ref.at[wid], idx_v, sem2)
            c3 = pltpu.async_copy(st_ref.at[n], st_v, sem3)
            c1.wait()
            c2.wait()
            c3.wait()
            for c in range(T // L):
                sl = pl.ds(c * L, L)
                tv = lax.broadcasted_iota(jnp.int32, (L,), 0) + c * L
                enc = plsc.load_gather(idx_v, [tv, tv * 0])
                hv = enc >= 0
                r3 = jnp.maximum(enc, 0)
                ax = plsc.load_gather(table_v, [r3])
                ay = plsc.load_gather(table_v, [r3 + 1])
                az = plsc.load_gather(table_v, [r3 + 2])
                bx = plsc.load_gather(table_v, [r3 + 3])
                by = plsc.load_gather(table_v, [r3 + 4])
                bz = plsc.load_gather(table_v, [r3 + 5])
                t6 = tv * F
                px = plsc.load_gather(st_v, [t6])
                py = plsc.load_gather(st_v, [t6 + 1])
                pz = plsc.load_gather(st_v, [t6 + 2])
                dx = bx - ax
                dy = by - ay
                dz = bz - az
                dd = jnp.maximum(dx * dx + dy * dy + dz * dz, 1e-12)
                tn = (px - ax) * dx + (py - ay) * dy + (pz - az) * dz
                t = jnp.clip(tn / dd, 0.0, 1.0)
                qx = ax + t * dx
                qy = ay + t * dy
                qz = az + t * dz
                plsc.store_scatter(st_v, [t6], jnp.where(hv, qx, px))
                plsc.store_scatter(st_v, [t6 + 1], jnp.where(hv, qy, py))
                plsc.store_scatter(st_v, [t6 + 2], jnp.where(hv, qz, pz))
            pltpu.sync_copy(st_v, out_ref.at[wid])

    return _sc_kernel


def _sc_run(rp_flat, rows, st_flat, off):
    NC = rows.shape[0]
    T = rows.shape[1]
    F = st_flat.shape[1] // T
    mesh = plsc.VectorSubcoreMesh(core_axis_name="c", subcore_axis_name="s",
                                  num_cores=2, num_subcores=16)
    cp = pltpu.CompilerParams()
    if "needs_layout_passes" in pltpu.CompilerParams.__dataclass_fields__:
        cp = dataclasses.replace(cp, needs_layout_passes=False)
    k = pl.kernel(
        _make_sc_kernel(T, F, off),
        out_type=jax.ShapeDtypeStruct((NC, T * F), jnp.float32),
        mesh=mesh,
        scratch_types=[
            pltpu.VMEM((rp_flat.shape[1],), jnp.float32),
            pltpu.VMEM((T, 1), jnp.int32),
            pltpu.VMEM((T * F,), jnp.float32),
            pltpu.SemaphoreType.DMA,
            pltpu.SemaphoreType.DMA,
            pltpu.SemaphoreType.DMA,
        ],
        compiler_params=cp,
    )
    return k(rp_flat, rows, st_flat)


@jax.jit
def _run(selected_traj, road_points, road_mask):
    N, T, F = selected_traj.shape
    _, NB, NP, D = road_points.shape
    NSP = NB * NP

    st = selected_traj.astype(jnp.float32)
    rpT = road_points.transpose(0, 3, 1, 2).reshape(N, D, NSP)
    rp_flat = road_points.reshape(N, NSP * D)
    rmf = road_mask.reshape(N, 1, NSP).astype(jnp.float32)

    st_flat = st.reshape(N, T * F)
    half = N // 2
    rows0 = _tc_run(st, rpT, rmf, 0, half)
    rows1 = _tc_run(st, rpT, rmf, half, half)
    out0 = _sc_run(rp_flat, rows0, st_flat, 0)
    out1 = _sc_run(rp_flat, rows1, st_flat, half)
    return jnp.concatenate([out0, out1], axis=0).reshape(N, T, F)


def kernel(selected_traj, road_points, road_mask):
    return _run(selected_traj, road_points, road_mask)


# speedup vs baseline: 1.3460x; 1.1245x over previous
import dataclasses
import functools

import jax
import jax.numpy as jnp
from jax import lax
from jax.experimental import pallas as pl
from jax.experimental.pallas import tpu as pltpu
from jax.experimental.pallas import tpu_sc as plsc


_ROWS_PER_STEP = 2


def _tc_kernel(st_ref, a_ref, rmf_ref, rows_ref):
    NSP = a_ref.shape[2]
    T = st_ref.shape[1]
    lane = jax.lax.broadcasted_iota(jnp.int32, (1, NSP), 1)
    lane_ok = jnp.where((lane & 127) != 127, 1.0, 0.0)
    z1 = jnp.zeros((1, 1), jnp.float32)

    for i in range(_ROWS_PER_STEP):
        p = st_ref[i]
        P3 = p[:, 0:3]
        ax = a_ref[i, 0:1, :]
        ay = a_ref[i, 1:2, :]
        az = a_ref[i, 2:3, :]
        rmf = rmf_ref[i]

        bx = jnp.concatenate([ax[:, 1:], z1], axis=1)
        by = jnp.concatenate([ay[:, 1:], z1], axis=1)
        bz = jnp.concatenate([az[:, 1:], z1], axis=1)
        dx = bx - ax
        dy = by - ay
        dz = bz - az
        dd = jnp.maximum(dx * dx + dy * dy + dz * dz, 1e-12)
        hrdd = 0.5 / dd
        rr = ax * ax + ay * ay + az * az
        rmf_next = jnp.concatenate([rmf[:, 1:], z1], axis=1)
        m = rmf * rmf_next * lane_ok
        mbig = (1.0 - m) * 1e30

        A3 = jnp.concatenate([ax, ay, az], axis=0)
        ph = P3.astype(jnp.bfloat16)
        plo = (P3 - ph.astype(jnp.float32)).astype(jnp.bfloat16)
        ah = A3.astype(jnp.bfloat16)
        alo = (A3 - ah.astype(jnp.float32)).astype(jnp.bfloat16)
        dn = (((1,), (0,)), ((), ()))
        mm = lambda x, y: jax.lax.dot_general(
            x, y, dimension_numbers=dn,
            preferred_element_type=jnp.float32)
        pr = ((mm(ph, ah) + mm(plo, alo))
              + (mm(ph, alo) + mm(plo, ah)))
        G = rr - (pr + pr)
        Gnext = jnp.concatenate([G[:, 1:], jnp.zeros((T, 1), jnp.float32)],
                                axis=1)
        u = (G - Gnext) * hrdd + 0.5
        t = jnp.clip(u, 0.0, 1.0)
        s = t - u - u
        dist2 = G + dd * (t * s) + mbig

        best = jnp.argmin(dist2, axis=1).astype(jnp.int32)
        hv = jnp.any(m > 0.0)
        enc = jnp.where(hv, best, -1)
        rows_ref[i] = enc[:, None]


def _tc_run(st, rpT, rmf, off, rows):
    N, T, F = st.shape
    NSP = rpT.shape[2]
    R = _ROWS_PER_STEP
    base = off // R
    return pl.pallas_call(
        _tc_kernel,
        grid=(rows // R,),
        in_specs=[
            pl.BlockSpec((R, T, F), lambda n: (n + base, 0, 0)),
            pl.BlockSpec((R, 3, NSP), lambda n: (n + base, 0, 0)),
            pl.BlockSpec((R, 1, NSP), lambda n: (n + base, 0, 0)),
        ],
        out_specs=pl.BlockSpec((R, T, 1), lambda n: (n, 0, 0)),
        out_shape=jax.ShapeDtypeStruct((rows, T, 1), jnp.int32),
    )(st, rpT, rmf)


def _make_sc_kernel(T, F, off):
    L = 16

    def _sc_kernel(rp_ref, rows_ref, st_ref, out_ref,
                   table_v, idx_v, st_v, sem1, sem2, sem3):
        wid = lax.axis_index("s") * 2 + lax.axis_index("c")
        if True:
            n = wid + off
            c1 = pltpu.async_copy(rp_ref.at[n], table_v, sem1)
            c2 = pltpu.async_copy(rows_ref.at[wid], idx_v, sem2)
            c3 = pltpu.async_copy(st_ref.at[n], st_v, sem3)
            c1.wait()
            c2.wait()
            c3.wait()
            for c in range(T // L):
                tv = lax.broadcasted_iota(jnp.int32, (L,), 0) + c * L
                zv = tv * 0
                ov = zv + 1
                wv = zv + 2
                enc = plsc.load_gather(idx_v, [tv, zv])
                hv = enc >= 0
                r = jnp.maximum(enc, 0)
                ax = plsc.load_gather(table_v, [zv, r])
                ay = plsc.load_gather(table_v, [ov, r])
                az = plsc.load_gather(table_v, [wv, r])
                bx = plsc.load_gather(table_v, [zv, r + 1])
                by = plsc.load_gather(table_v, [ov, r + 1])
                bz = plsc.load_gather(table_v, [wv, r + 1])
                px = plsc.load_gather(st_v, [tv, zv])
                py = plsc.load_gather(st_v, [tv, ov])
                pz = plsc.load_gather(st_v, [tv, wv])
                dx = bx - ax
                dy = by - ay
                dz = bz - az
                dd = jnp.maximum(dx * dx + dy * dy + dz * dz, 1e-12)
                tn = (px - ax) * dx + (py - ay) * dy + (pz - az) * dz
                t = jnp.clip(tn / dd, 0.0, 1.0)
                qx = ax + t * dx
                qy = ay + t * dy
                qz = az + t * dz
                plsc.store_scatter(st_v, [tv, zv], jnp.where(hv, qx, px))
                plsc.store_scatter(st_v, [tv, ov], jnp.where(hv, qy, py))
                plsc.store_scatter(st_v, [tv, wv], jnp.where(hv, qz, pz))
            pltpu.sync_copy(st_v, out_ref.at[wid])

    return _sc_kernel


def _sc_run(rpT, rows, st, off):
    NC = rows.shape[0]
    T = rows.shape[1]
    F = st.shape[2]
    NSP = rpT.shape[2]
    mesh = plsc.VectorSubcoreMesh(core_axis_name="c", subcore_axis_name="s",
                                  num_cores=2, num_subcores=16)
    cp = pltpu.CompilerParams()
    if "needs_layout_passes" in pltpu.CompilerParams.__dataclass_fields__:
        cp = dataclasses.replace(cp, needs_layout_passes=False)
    k = pl.kernel(
        _make_sc_kernel(T, F, off),
        out_type=jax.ShapeDtypeStruct((NC, T, F), jnp.float32),
        mesh=mesh,
        scratch_types=[
            pltpu.VMEM((3, NSP), jnp.float32),
            pltpu.VMEM((T, 1), jnp.int32),
            pltpu.VMEM((T, F), jnp.float32),
            pltpu.SemaphoreType.DMA,
            pltpu.SemaphoreType.DMA,
            pltpu.SemaphoreType.DMA,
        ],
        compiler_params=cp,
    )
    return k(rpT, rows, st)


@jax.jit
def _run(selected_traj, road_points, road_mask):
    N, T, F = selected_traj.shape
    _, NB, NP, D = road_points.shape
    NSP = NB * NP

    st = selected_traj.astype(jnp.float32)
    rpT = road_points.transpose(0, 3, 1, 2).reshape(N, D, NSP)
    rmf = road_mask.reshape(N, 1, NSP).astype(jnp.float32)

    half = N // 2
    rows0 = _tc_run(st, rpT, rmf, 0, half)
    rows1 = _tc_run(st, rpT, rmf, half, half)
    out0 = _sc_run(rpT, rows0, st, 0)
    out1 = _sc_run(rpT, rows1, st, half)
    return jnp.concatenate([out0, out1], axis=0)


def kernel(selected_traj, road_points, road_mask):
    return _run(selected_traj, road_points, road_mask)


# speedup vs baseline: 1.3473x; 1.0010x over previous
import dataclasses

import jax
import jax.numpy as jnp
from jax import lax
from jax.experimental import pallas as pl
from jax.experimental.pallas import tpu as pltpu
from jax.experimental.pallas import tpu_sc as plsc


_ROWS_PER_STEP = 2


def _tc_kernel(st_ref, a_ref, rmf_ref, rows_ref):
    NSP = a_ref.shape[2]
    T = st_ref.shape[1]
    lane = jax.lax.broadcasted_iota(jnp.int32, (1, NSP), 1)
    lane_ok = jnp.where((lane & 127) != 127, 1.0, 0.0)
    z1 = jnp.zeros((1, 1), jnp.float32)

    for i in range(_ROWS_PER_STEP):
        p = st_ref[i]
        P3 = p[:, 0:3]
        ax = a_ref[i, 0:1, :]
        ay = a_ref[i, 1:2, :]
        az = a_ref[i, 2:3, :]
        rmf = rmf_ref[i]

        bx = jnp.concatenate([ax[:, 1:], z1], axis=1)
        by = jnp.concatenate([ay[:, 1:], z1], axis=1)
        bz = jnp.concatenate([az[:, 1:], z1], axis=1)
        dx = bx - ax
        dy = by - ay
        dz = bz - az
        dd = jnp.maximum(dx * dx + dy * dy + dz * dz, 1e-12)
        hrdd = 0.5 / dd
        rr = ax * ax + ay * ay + az * az
        rmf_next = jnp.concatenate([rmf[:, 1:], z1], axis=1)
        m = rmf * rmf_next * lane_ok
        mbig = (1.0 - m) * 1e30

        A3 = jnp.concatenate([ax, ay, az], axis=0)
        ph = P3.astype(jnp.bfloat16)
        plo = (P3 - ph.astype(jnp.float32)).astype(jnp.bfloat16)
        ah = A3.astype(jnp.bfloat16)
        alo = (A3 - ah.astype(jnp.float32)).astype(jnp.bfloat16)
        dn = (((1,), (0,)), ((), ()))
        mm = lambda x, y: jax.lax.dot_general(
            x, y, dimension_numbers=dn,
            preferred_element_type=jnp.float32)
        pr = ((mm(ph, ah) + mm(plo, alo))
              + (mm(ph, alo) + mm(plo, ah)))
        G = rr - (pr + pr)
        Gnext = jnp.concatenate([G[:, 1:], jnp.zeros((T, 1), jnp.float32)],
                                axis=1)
        u = (G - Gnext) * hrdd + 0.5
        t = jnp.clip(u, 0.0, 1.0)
        s = t - u - u
        dist2 = G + dd * (t * s) + mbig

        best = jnp.argmin(dist2, axis=1).astype(jnp.int32)
        hv = jnp.any(m > 0.0)
        enc = jnp.where(hv, best, -1)
        rows_ref[i] = enc[:, None]


def _tc_run(st, rpT, rmf, off, rows):
    N, T, F = st.shape
    NSP = rpT.shape[2]
    R = _ROWS_PER_STEP
    base = off // R
    return pl.pallas_call(
        _tc_kernel,
        grid=(rows // R,),
        in_specs=[
            pl.BlockSpec((R, T, F), lambda n: (n + base, 0, 0)),
            pl.BlockSpec((R, 3, NSP), lambda n: (n + base, 0, 0)),
            pl.BlockSpec((R, 1, NSP), lambda n: (n + base, 0, 0)),
        ],
        out_specs=pl.BlockSpec((R, T, 1), lambda n: (n, 0, 0)),
        out_shape=jax.ShapeDtypeStruct((rows, T, 1), jnp.int32),
    )(st, rpT, rmf)


def _make_sc_kernel(T, F, off):
    L = 16

    def _sc_kernel(rp_ref, rows_ref, st_ref, out_ref,
                   table_v, idx_v, st_v, sem1, sem2, sem3):
        wid = lax.axis_index("s") * 2 + lax.axis_index("c")
        n = wid + off
        c1 = pltpu.async_copy(rp_ref.at[n], table_v, sem1)
        c2 = pltpu.async_copy(rows_ref.at[wid], idx_v, sem2)
        c3 = pltpu.async_copy(st_ref.at[n], st_v, sem3)
        c1.wait()
        c2.wait()
        c3.wait()
        for c in range(T // L):
            tv = lax.broadcasted_iota(jnp.int32, (L,), 0) + c * L
            zv = tv * 0
            ov = zv + 1
            wv = zv + 2
            enc = plsc.load_gather(idx_v, [tv, zv])
            hv = enc >= 0
            r = jnp.maximum(enc, 0)
            ax = plsc.load_gather(table_v, [zv, r])
            ay = plsc.load_gather(table_v, [ov, r])
            az = plsc.load_gather(table_v, [wv, r])
            bx = plsc.load_gather(table_v, [zv, r + 1])
            by = plsc.load_gather(table_v, [ov, r + 1])
            bz = plsc.load_gather(table_v, [wv, r + 1])
            px = plsc.load_gather(st_v, [tv, zv])
            py = plsc.load_gather(st_v, [tv, ov])
            pz = plsc.load_gather(st_v, [tv, wv])
            dx = bx - ax
            dy = by - ay
            dz = bz - az
            dd = jnp.maximum(dx * dx + dy * dy + dz * dz, 1e-12)
            tn = (px - ax) * dx + (py - ay) * dy + (pz - az) * dz
            t = jnp.clip(tn / dd, 0.0, 1.0)
            qx = ax + t * dx
            qy = ay + t * dy
            qz = az + t * dz
            plsc.store_scatter(st_v, [tv, zv], jnp.where(hv, qx, px))
            plsc.store_scatter(st_v, [tv, ov], jnp.where(hv, qy, py))
            plsc.store_scatter(st_v, [tv, wv], jnp.where(hv, qz, pz))
        pltpu.sync_copy(st_v, out_ref.at[wid])

    return _sc_kernel


def _sc_run(rpT, rows, st, off):
    NC = rows.shape[0]
    T = rows.shape[1]
    F = st.shape[2]
    NSP = rpT.shape[2]
    mesh = plsc.VectorSubcoreMesh(core_axis_name="c", subcore_axis_name="s",
                                  num_cores=2, num_subcores=16)
    cp = pltpu.CompilerParams()
    if "needs_layout_passes" in pltpu.CompilerParams.__dataclass_fields__:
        cp = dataclasses.replace(cp, needs_layout_passes=False)
    k = pl.kernel(
        _make_sc_kernel(T, F, off),
        out_type=jax.ShapeDtypeStruct((NC, T, F), jnp.float32),
        mesh=mesh,
        scratch_types=[
            pltpu.VMEM((3, NSP), jnp.float32),
            pltpu.VMEM((T, 1), jnp.int32),
            pltpu.VMEM((T, F), jnp.float32),
            pltpu.SemaphoreType.DMA,
            pltpu.SemaphoreType.DMA,
            pltpu.SemaphoreType.DMA,
        ],
        compiler_params=cp,
    )
    return k(rpT, rows, st)


@jax.jit
def _run(selected_traj, road_points, road_mask):
    N, T, F = selected_traj.shape
    _, NB, NP, D = road_points.shape
    NSP = NB * NP

    st = selected_traj.astype(jnp.float32)
    rpT = road_points.transpose(0, 3, 1, 2).reshape(N, D, NSP)
    rmf = road_mask.reshape(N, 1, NSP).astype(jnp.float32)

    half = N // 2
    rows0 = _tc_run(st, rpT, rmf, 0, half)
    rows1 = _tc_run(st, rpT, rmf, half, half)
    out0 = _sc_run(rpT, rows0, st, 0)
    out1 = _sc_run(rpT, rows1, st, half)
    return jnp.concatenate([out0, out1], axis=0)


def kernel(selected_traj, road_points, road_mask):
    return _run(selected_traj, road_points, road_mask)
